# baseline (speedup 1.0000x reference)
import functools

import jax
import jax.numpy as jnp
from jax import lax
from jax.experimental import pallas as pl
from jax.experimental.pallas import tpu as pltpu
from jax.experimental.pallas import tpu_sc as plsc

_LANES = 16
_IDX_CHUNK = 128


@functools.cache
def _build(batch: int, feat: int, num_classes: int):
    info = plsc.get_sparse_core_info()
    nc, ns = info.num_cores, info.num_subcores
    nw = nc * ns
    assert batch % (nw * _IDX_CHUNK) == 0
    b_per_w = batch // nw
    n_chunks = b_per_w // _IDX_CHUNK
    feat_vecs = feat // _LANES
    scale = 1.0 / (batch * feat)

    mesh = plsc.VectorSubcoreMesh(core_axis_name="c", subcore_axis_name="s")

    @functools.partial(
        pl.kernel,
        mesh=mesh,
        out_type=jax.ShapeDtypeStruct((nw, _LANES), jnp.float32),
        scratch_types=[
            pltpu.VMEM((b_per_w,), jnp.int32),
            pltpu.VMEM((b_per_w, feat), jnp.float32),
            pltpu.VMEM((b_per_w, feat), jnp.float32),
            pltpu.VMEM((_LANES,), jnp.float32),
            pltpu.SemaphoreType.DMA,
        ],
        compiler_params=pltpu.CompilerParams(use_tc_tiling_on_sc=False),
    )
    def k(x_hbm, labels_hbm, centers_hbm, out_hbm, idx_v, xs_v, rows_v, acc_v, sem):
        wid = lax.axis_index("s") * nc + lax.axis_index("c")
        base = wid * b_per_w
        pltpu.sync_copy(labels_hbm.at[pl.ds(base, b_per_w)], idx_v)
        pltpu.sync_copy(x_hbm.at[pl.ds(base, b_per_w)], xs_v)
        for c in range(n_chunks):
            pltpu.async_copy(
                centers_hbm.at[idx_v.at[pl.ds(c * _IDX_CHUNK, _IDX_CHUNK)]],
                rows_v.at[pl.ds(c * _IDX_CHUNK, _IDX_CHUNK)],
                sem,
            ).wait()

        def body(r, acc):
            for f in range(feat_vecs):
                dx = xs_v[r, pl.ds(f * _LANES, _LANES)] - rows_v[r, pl.ds(f * _LANES, _LANES)]
                acc = acc + dx * dx
            return acc

        acc = lax.fori_loop(0, b_per_w, body, jnp.zeros((_LANES,), jnp.float32))
        acc_v[...] = acc * scale
        pltpu.sync_copy(acc_v, out_hbm.at[wid])

    return k


def kernel(x, labels, centers):
    batch, feat = x.shape
    k = _build(batch, feat, centers.shape[0])
    partials = k(x, labels.astype(jnp.int32), centers)
    return jnp.sum(partials)


# speedup vs baseline: 1.3206x; 1.3206x over previous
import functools

import jax
import jax.numpy as jnp
from jax import lax
from jax.experimental import pallas as pl
from jax.experimental.pallas import tpu as pltpu
from jax.experimental.pallas import tpu_sc as plsc

_LANES = 16
_SUB = 8


@functools.cache
def _build(batch: int, feat: int, num_classes: int):
    info = plsc.get_sparse_core_info()
    nc, ns = info.num_cores, info.num_subcores
    nw = nc * ns
    assert batch % (nw * _SUB * _LANES) == 0 and feat % _LANES == 0
    b_per_w = batch // nw
    n_groups = b_per_w // _LANES
    n_tiles = b_per_w // _SUB
    feat_vecs = feat // _LANES
    rows_per_line = 128 // feat
    scale = 1.0 / (batch * feat)

    mesh = plsc.VectorSubcoreMesh(core_axis_name="c", subcore_axis_name="s")

    @functools.partial(
        pl.kernel,
        mesh=mesh,
        out_type=jax.ShapeDtypeStruct((nw, _LANES), jnp.float32),
        scratch_types=[
            pltpu.VMEM((b_per_w,), jnp.int32),
            pltpu.VMEM((n_tiles, _SUB, feat), jnp.float32),
            pltpu.VMEM((n_groups, _SUB, 128), jnp.float32),
            pltpu.VMEM((_LANES,), jnp.float32),
            pltpu.SemaphoreType.DMA,
        ],
    )
    def k(x_hbm, labels_hbm, centers_hbm, out_hbm,
          idx_v, xs_v, rows_v, acc_v, sem):
        wid = lax.axis_index("s") * nc + lax.axis_index("c")
        base = wid * b_per_w
        pltpu.sync_copy(labels_hbm.at[pl.ds(base, b_per_w)], idx_v)

        def issue(g, carry):
            v16 = idx_v[pl.ds(g * _LANES, _LANES)]
            for i in range(_LANES):
                lab = v16[i]
                pltpu.make_async_copy(
                    centers_hbm.at[lab >> 3, lab & 7],
                    rows_v.at[g, i // rows_per_line,
                              pl.ds((i % rows_per_line) * feat, feat)],
                    sem,
                ).start()
            return carry

        def drain_group(g, carry):
            for i in range(_LANES):
                pltpu.make_async_copy(
                    centers_hbm.at[0, 0],
                    rows_v.at[0, 0, pl.ds(0, feat)],
                    sem,
                ).wait()
            return carry

        pltpu.sync_copy(x_hbm.at[pl.ds(base // _SUB, n_tiles)], xs_v)

        def step(g, carry):
            issue(g, 0)
            drain_group(g, 0)
            return carry

        lax.fori_loop(0, n_groups, step, 0)

        def body(g, acc):
            for i in range(_LANES):
                xt = 2 * g + i // _SUB
                xb = i % _SUB
                rs = i // rows_per_line
                rc = (i % rows_per_line) * feat
                for f in range(feat_vecs):
                    dx = (xs_v[xt, xb, pl.ds(f * _LANES, _LANES)]
                          - rows_v[g, rs, pl.ds(rc + f * _LANES, _LANES)])
                    acc = acc + dx * dx
            return acc

        acc = lax.fori_loop(0, n_groups, body, jnp.zeros((_LANES,), jnp.float32))
        acc_v[...] = acc * scale
        pltpu.sync_copy(acc_v, out_hbm.at[wid])

    return k


def kernel(x, labels, centers):
    batch, feat = x.shape
    num_classes = centers.shape[0]
    k = _build(batch, feat, num_classes)
    x3 = x.reshape(batch // _SUB, _SUB, feat)
    c3 = centers.reshape(num_classes // _SUB, _SUB, feat)
    partials = k(x3, labels.astype(jnp.int32), c3)
    return jnp.sum(partials)


# speedup vs baseline: 1.6696x; 1.2643x over previous
import functools

import jax
import jax.numpy as jnp
from jax import lax
from jax.experimental import pallas as pl
from jax.experimental.pallas import tpu as pltpu
from jax.experimental.pallas import tpu_sc as plsc

_LANES = 16
_SUB = 8


@functools.cache
def _build(batch: int, feat: int, num_classes: int):
    info = plsc.get_sparse_core_info()
    nc, ns = info.num_cores, info.num_subcores
    nw = nc * ns
    assert batch % (nw * _SUB * _LANES) == 0 and feat % _LANES == 0
    b_per_w = batch // nw
    n_groups = b_per_w // _LANES
    n_tiles = b_per_w // _SUB
    feat_vecs = feat // _LANES
    rows_per_line = 128 // feat
    scale = 1.0 / (batch * feat)

    mesh = plsc.VectorSubcoreMesh(core_axis_name="c", subcore_axis_name="s")

    @functools.partial(
        pl.kernel,
        mesh=mesh,
        out_type=jax.ShapeDtypeStruct((nw, _LANES), jnp.float32),
        scratch_types=[
            pltpu.VMEM((b_per_w,), jnp.int32),
            pltpu.VMEM((n_tiles, _SUB, feat), jnp.float32),
            pltpu.VMEM((n_groups, _SUB, 128), jnp.float32),
            pltpu.VMEM((_LANES,), jnp.float32),
            pltpu.SemaphoreType.DMA,
        ],
    )
    def k(x_hbm, labels_hbm, centers_hbm, out_hbm,
          idx_v, xs_v, rows_v, acc_v, sem):
        wid = lax.axis_index("s") * nc + lax.axis_index("c")
        base = wid * b_per_w
        pltpu.sync_copy(labels_hbm.at[pl.ds(base, b_per_w)], idx_v)

        def issue(g, carry):
            v16 = idx_v[pl.ds(g * _LANES, _LANES)]
            for i in range(_LANES):
                lab = v16[i]
                pltpu.make_async_copy(
                    centers_hbm.at[lab >> 3, lab & 7],
                    rows_v.at[g, i // rows_per_line,
                              pl.ds((i % rows_per_line) * feat, feat)],
                    sem,
                ).start()
            return carry

        def drain_group(g, carry):
            for i in range(_LANES):
                pltpu.make_async_copy(
                    centers_hbm.at[0, 0],
                    rows_v.at[0, 0, pl.ds(0, feat)],
                    sem,
                ).wait()
            return carry

        depth = 8
        lax.fori_loop(0, depth, issue, 0)
        pltpu.sync_copy(x_hbm.at[pl.ds(base // _SUB, n_tiles)], xs_v)

        def step(g, carry):
            issue(g, 0)
            drain_group(g, 0)
            return carry

        lax.fori_loop(depth, n_groups, step, 0)
        lax.fori_loop(0, depth, drain_group, 0)

        def body(g, acc):
            for i in range(_LANES):
                xt = 2 * g + i // _SUB
                xb = i % _SUB
                rs = i // rows_per_line
                rc = (i % rows_per_line) * feat
                for f in range(feat_vecs):
                    dx = (xs_v[xt, xb, pl.ds(f * _LANES, _LANES)]
                          - rows_v[g, rs, pl.ds(rc + f * _LANES, _LANES)])
                    acc = acc + dx * dx
            return acc

        acc = lax.fori_loop(0, n_groups, body, jnp.zeros((_LANES,), jnp.float32))
        acc_v[...] = acc * scale
        pltpu.sync_copy(acc_v, out_hbm.at[wid])

    return k


def kernel(x, labels, centers):
    batch, feat = x.shape
    num_classes = centers.shape[0]
    k = _build(batch, feat, num_classes)
    x3 = x.reshape(batch // _SUB, _SUB, feat)
    c3 = centers.reshape(num_classes // _SUB, _SUB, feat)
    partials = k(x3, labels.astype(jnp.int32), c3)
    return jnp.sum(partials)
